# baseline (speedup 1.0000x reference)
import functools

import jax
import jax.numpy as jnp
from jax import lax
from jax.experimental import pallas as pl
from jax.experimental.pallas import tpu as pltpu
from jax.experimental.pallas import tpu_sc as plsc

N_NODES = 10000
D = 128
N_EDGES = 320000

NC = 2
NS = 16
NW = NC * NS
CHUNK = 128
CPT = 80
NBUF = 4
E_PAD = NW * CPT * CHUNK
N_PAD = 10240
STRIPE = N_PAD // NS

_MESH = plsc.VectorSubcoreMesh(core_axis_name="c", subcore_axis_name="s")


def _zero_vmem_rows(ref, nrows):
    zrow = jnp.zeros((16,), jnp.float32)

    def zr(i, c):
        for k in range(D // 16):
            ref[i, pl.ds(k * 16, 16)] = zrow
        return c

    lax.fori_loop(0, nrows, zr, 0)


@functools.partial(
    pl.kernel,
    out_type=jax.ShapeDtypeStruct((NC, N_PAD), jnp.float32),
    mesh=_MESH,
    scratch_types=[
        pltpu.VMEM((CPT, CHUNK), jnp.int32),
        pltpu.VMEM((CPT, CHUNK), jnp.float32),
        pltpu.VMEM((STRIPE,), jnp.float32),
        pltpu.VMEM_SHARED((N_PAD,), jnp.float32),
        pltpu.SemaphoreType.DMA,
    ],
)
def _sc_degree(dst_hbm, w_hbm, out_hbm, dst_v, w_v, z_v, acc, sem):
    cid = lax.axis_index("c")
    sid = lax.axis_index("s")
    wid = cid * NS + sid

    zrow = jnp.zeros((16,), jnp.float32)
    for k in range(STRIPE // 16):
        z_v[pl.ds(k * 16, 16)] = zrow
    pltpu.sync_copy(z_v, acc.at[pl.ds(sid * STRIPE, STRIPE)])
    pltpu.sync_copy(dst_hbm.at[pl.ds(wid * CPT, CPT)], dst_v)
    pltpu.sync_copy(w_hbm.at[pl.ds(wid * CPT, CPT)], w_v)
    plsc.subcore_barrier()

    def fire(c, carry):
        pltpu.async_copy(w_v.at[c], acc.at[dst_v.at[c]], sem, add=True)
        return carry

    lax.fori_loop(0, CPT, fire, 0)

    def drain(c, carry):
        pltpu.make_async_copy(w_v.at[0], acc.at[pl.ds(0, CHUNK)], sem).wait()
        return carry

    lax.fori_loop(0, CPT, drain, 0)

    plsc.subcore_barrier()
    pltpu.sync_copy(acc.at[pl.ds(sid * STRIPE, STRIPE)],
                    out_hbm.at[cid, pl.ds(sid * STRIPE, STRIPE)])


@functools.partial(
    pl.kernel,
    out_type=jax.ShapeDtypeStruct((NC, N_PAD, D), jnp.float32),
    mesh=_MESH,
    scratch_types=[
        pltpu.VMEM((4, 2, CHUNK), jnp.int32),
        pltpu.VMEM((4, CHUNK), jnp.float32),
        pltpu.VMEM((2, CHUNK, D), jnp.float32),
        pltpu.VMEM_SHARED((N_PAD, D), jnp.float32),
        [pltpu.SemaphoreType.DMA] * 2,
        [pltpu.SemaphoreType.DMA] * 2,
        [pltpu.SemaphoreType.DMA] * 2,
    ],
)
def _sc_aggregate(y_hbm, ed_hbm, w_hbm, out_hbm, ed_v, w_v, rows_v, acc,
                  esem, gsem, ssem):
    cid = lax.axis_index("c")
    sid = lax.axis_index("s")
    wid = cid * NS + sid
    r0 = sid * STRIPE
    ebase = wid * CPT

    _zero_vmem_rows(rows_v.at[0], CHUNK)
    for t in range(STRIPE // CHUNK):
        pltpu.sync_copy(rows_v.at[0], acc.at[pl.ds(r0 + t * CHUNK, CHUNK)])
    plsc.subcore_barrier()

    def fire_ed(t, e, s):
        pltpu.async_copy(ed_hbm.at[ebase + t], ed_v.at[e], esem[s])
        pltpu.async_copy(w_hbm.at[ebase + t], w_v.at[e], esem[s])

    def drain_ed(s):
        pltpu.make_async_copy(ed_hbm.at[0], ed_v.at[0], esem[s]).wait()
        pltpu.make_async_copy(w_hbm.at[0], w_v.at[0], esem[s]).wait()

    def fire_gather(e, b):
        pltpu.async_copy(y_hbm.at[ed_v.at[e, 0]], rows_v.at[b], gsem[b])

    def drain_gather(b):
        pltpu.make_async_copy(y_hbm.at[pl.ds(0, CHUNK)], rows_v.at[b],
                              gsem[b]).wait()

    def fire_scatter(e, b):
        pltpu.async_copy(rows_v.at[b], acc.at[ed_v.at[e, 1]], ssem[b],
                         add=True)

    def drain_scatter(b):
        pltpu.make_async_copy(rows_v.at[b], acc.at[pl.ds(0, CHUNK)],
                              ssem[b]).wait()

    fire_ed(0, 0, 0)
    fire_ed(1, 1, 1)
    drain_ed(0)
    fire_gather(0, 0)

    def rnd(g, carry):
        for b in range(4):
            t = g * 4 + b
            rb = b % 2
            drain_gather(rb)

            def sgrp(q, cc):
                wv = w_v[b, pl.ds(q * 16, 16)]
                for l in range(16):
                    s = wv[l]
                    r = q * 16 + l
                    for k in range(D // 16):
                        rows_v[rb, r, pl.ds(k * 16, 16)] = (
                            rows_v[rb, r, pl.ds(k * 16, 16)] * s)
                return cc

            lax.fori_loop(0, CHUNK // 16, sgrp, 0)
            fire_scatter(b, rb)

            @pl.when(t < CPT - 1)
            def _():
                @pl.when(t >= 1)
                def _():
                    drain_scatter((b + 1) % 2)
                drain_ed((b + 1) % 2)
                fire_gather((b + 1) % 4, (b + 1) % 2)

            @pl.when(t < CPT - 2)
            def _():
                fire_ed(t + 2, (b + 2) % 4, b % 2)
        return carry

    lax.fori_loop(0, CPT // 4, rnd, 0)
    drain_scatter(0)
    drain_scatter(1)

    plsc.subcore_barrier()
    for t in range(STRIPE // CHUNK):
        pltpu.sync_copy(acc.at[pl.ds(r0 + t * CHUNK, CHUNK)],
                        out_hbm.at[cid, pl.ds(r0 + t * CHUNK, CHUNK)])


_R = 400
_GRID = (N_NODES // _R,)
_INV_SQRT2 = 0.7071067811865476


def _gelu(t):
    return 0.5 * t * (1.0 + lax.erf(t * _INV_SQRT2))


def _pre_body(x_ref, w_ref, d0_ref, d1_ref, y_ref, dis_ref):
    deg = d0_ref[...] + d1_ref[...] + 1.0
    dis = lax.rsqrt(deg)
    xw = jnp.dot(x_ref[...], w_ref[...], preferred_element_type=jnp.float32)
    y_ref[...] = xw * dis
    dis_ref[...] = dis


def _tc_pre(x, W1, d0, d1):
    return pl.pallas_call(
        _pre_body,
        grid=_GRID,
        in_specs=[
            pl.BlockSpec((_R, D), lambda i: (i, 0)),
            pl.BlockSpec((D, D), lambda i: (0, 0)),
            pl.BlockSpec((_R, 1), lambda i: (i, 0)),
            pl.BlockSpec((_R, 1), lambda i: (i, 0)),
        ],
        out_specs=[
            pl.BlockSpec((_R, D), lambda i: (i, 0)),
            pl.BlockSpec((_R, 1), lambda i: (i, 0)),
        ],
        out_shape=[
            jax.ShapeDtypeStruct((N_NODES, D), jnp.float32),
            jax.ShapeDtypeStruct((N_NODES, 1), jnp.float32),
        ],
    )(x, W1, d0, d1)


def _mid_body(p0_ref, p1_ref, y_ref, dis_ref, b_ref, w2_ref, y2_ref):
    dis = dis_ref[...]
    t = dis * (p0_ref[...] + p1_ref[...] + y_ref[...]) + b_ref[...]
    h = _gelu(t)
    y2_ref[...] = jnp.dot(h, w2_ref[...],
                          preferred_element_type=jnp.float32) * dis


def _tc_mid(p0, p1, y1, dis, b1, W2):
    return pl.pallas_call(
        _mid_body,
        grid=_GRID,
        in_specs=[
            pl.BlockSpec((_R, D), lambda i: (i, 0)),
            pl.BlockSpec((_R, D), lambda i: (i, 0)),
            pl.BlockSpec((_R, D), lambda i: (i, 0)),
            pl.BlockSpec((_R, 1), lambda i: (i, 0)),
            pl.BlockSpec((1, D), lambda i: (0, 0)),
            pl.BlockSpec((D, D), lambda i: (0, 0)),
        ],
        out_specs=pl.BlockSpec((_R, D), lambda i: (i, 0)),
        out_shape=jax.ShapeDtypeStruct((N_NODES, D), jnp.float32),
    )(p0, p1, y1, dis, b1, W2)


def _post_body(q0_ref, q1_ref, y2_ref, dis_ref, b_ref, out_ref):
    t = dis_ref[...] * (q0_ref[...] + q1_ref[...] + y2_ref[...]) + b_ref[...]
    out_ref[...] = _gelu(t)


def _tc_post(q0, q1, y2, dis, b2):
    return pl.pallas_call(
        _post_body,
        grid=_GRID,
        in_specs=[
            pl.BlockSpec((_R, D), lambda i: (i, 0)),
            pl.BlockSpec((_R, D), lambda i: (i, 0)),
            pl.BlockSpec((_R, D), lambda i: (i, 0)),
            pl.BlockSpec((_R, 1), lambda i: (i, 0)),
            pl.BlockSpec((1, D), lambda i: (0, 0)),
        ],
        out_specs=pl.BlockSpec((_R, D), lambda i: (i, 0)),
        out_shape=jax.ShapeDtypeStruct((N_NODES, D), jnp.float32),
    )(q0, q1, y2, dis, b2)


def kernel(x, edge_index, edge_weight, W1, b1, W2, b2):
    src = edge_index[0].astype(jnp.int32)
    dst = edge_index[1].astype(jnp.int32)
    w = edge_weight.astype(jnp.float32)
    pad = E_PAD - N_EDGES
    src = jnp.concatenate([src, jnp.zeros((pad,), jnp.int32)])
    dst = jnp.concatenate([dst, jnp.zeros((pad,), jnp.int32)])
    w = jnp.concatenate([w, jnp.zeros((pad,), jnp.float32)])
    src = src.reshape(NW * CPT, CHUNK)
    dst = dst.reshape(NW * CPT, CHUNK)
    w = w.reshape(NW * CPT, CHUNK)
    ed = jnp.stack([src, dst], axis=1)

    degp = _sc_degree(dst, w)
    d0 = degp[0, :, None]
    d1 = degp[1, :, None]

    y1, dis = _tc_pre(x, W1, d0, d1)

    p = _sc_aggregate(y1, ed, w)
    y2 = _tc_mid(p[0], p[1], y1, dis,
                 b1.reshape(1, D), W2)

    q = _sc_aggregate(y2, ed, w)
    return _tc_post(q[0], q[1], y2, dis, b2.reshape(1, D))


# speedup vs baseline: 2.5267x; 2.5267x over previous
import functools

import jax
import jax.numpy as jnp
from jax import lax
from jax.experimental import pallas as pl
from jax.experimental.pallas import tpu as pltpu
from jax.experimental.pallas import tpu_sc as plsc

N_NODES = 10000
D = 128
N_EDGES = 320000

NC = 2
NS = 16
NW = NC * NS
CHUNK = 128
CPT = 80
NBUF = 4
E_PAD = NW * CPT * CHUNK
N_PAD = 10240
STRIPE = N_PAD // NS

_MESH = plsc.VectorSubcoreMesh(core_axis_name="c", subcore_axis_name="s")


def _zero_vmem_rows(ref, nrows):
    zrow = jnp.zeros((16,), jnp.float32)

    def zr(i, c):
        for k in range(D // 16):
            ref[i, pl.ds(k * 16, 16)] = zrow
        return c

    lax.fori_loop(0, nrows, zr, 0)


@functools.partial(
    pl.kernel,
    out_type=jax.ShapeDtypeStruct((NC, N_PAD), jnp.float32),
    mesh=_MESH,
    scratch_types=[
        pltpu.VMEM((CPT, CHUNK), jnp.int32),
        pltpu.VMEM((CPT, CHUNK), jnp.float32),
        pltpu.VMEM((STRIPE,), jnp.float32),
        pltpu.VMEM_SHARED((N_PAD,), jnp.float32),
        pltpu.SemaphoreType.DMA,
    ],
)
def _sc_degree(dst_hbm, w_hbm, out_hbm, dst_v, w_v, z_v, acc, sem):
    cid = lax.axis_index("c")
    sid = lax.axis_index("s")
    wid = cid * NS + sid

    zrow = jnp.zeros((16,), jnp.float32)
    for k in range(STRIPE // 16):
        z_v[pl.ds(k * 16, 16)] = zrow
    pltpu.sync_copy(z_v, acc.at[pl.ds(sid * STRIPE, STRIPE)])
    pltpu.sync_copy(dst_hbm.at[pl.ds(wid * CPT, CPT)], dst_v)
    pltpu.sync_copy(w_hbm.at[pl.ds(wid * CPT, CPT)], w_v)
    plsc.subcore_barrier()

    def fire(c, carry):
        pltpu.async_copy(w_v.at[c], acc.at[dst_v.at[c]], sem, add=True)
        return carry

    lax.fori_loop(0, CPT, fire, 0)

    def drain(c, carry):
        pltpu.make_async_copy(w_v.at[0], acc.at[pl.ds(0, CHUNK)], sem).wait()
        return carry

    lax.fori_loop(0, CPT, drain, 0)

    plsc.subcore_barrier()
    pltpu.sync_copy(acc.at[pl.ds(sid * STRIPE, STRIPE)],
                    out_hbm.at[cid, pl.ds(sid * STRIPE, STRIPE)])


@functools.partial(
    pl.kernel,
    out_type=jax.ShapeDtypeStruct((NC, N_PAD, D), jnp.float32),
    mesh=_MESH,
    scratch_types=[
        pltpu.VMEM((4, 2, CHUNK), jnp.int32),
        pltpu.VMEM((4, CHUNK), jnp.float32),
        pltpu.VMEM((2, CHUNK, D), jnp.float32),
        pltpu.VMEM_SHARED((N_PAD, D), jnp.float32),
        [pltpu.SemaphoreType.DMA] * 2,
        [pltpu.SemaphoreType.DMA] * 2,
        [pltpu.SemaphoreType.DMA] * 2,
    ],
)
def _sc_aggregate(y_hbm, ed_hbm, w_hbm, out_hbm, ed_v, w_v, rows_v, acc,
                  esem, gsem, ssem):
    cid = lax.axis_index("c")
    sid = lax.axis_index("s")
    wid = cid * NS + sid
    r0 = sid * STRIPE
    ebase = wid * CPT

    _zero_vmem_rows(rows_v.at[0], CHUNK)
    for t in range(STRIPE // CHUNK):
        pltpu.sync_copy(rows_v.at[0], acc.at[pl.ds(r0 + t * CHUNK, CHUNK)])
    plsc.subcore_barrier()

    def fire_ed(t, e, s):
        pltpu.async_copy(ed_hbm.at[ebase + t], ed_v.at[e], esem[s])
        pltpu.async_copy(w_hbm.at[ebase + t], w_v.at[e], esem[s])

    def drain_ed(s):
        pltpu.make_async_copy(ed_hbm.at[0], ed_v.at[0], esem[s]).wait()
        pltpu.make_async_copy(w_hbm.at[0], w_v.at[0], esem[s]).wait()

    def fire_gather(e, b):
        pltpu.async_copy(y_hbm.at[ed_v.at[e, 0]], rows_v.at[b], gsem[b])

    def drain_gather(b):
        pltpu.make_async_copy(y_hbm.at[pl.ds(0, CHUNK)], rows_v.at[b],
                              gsem[b]).wait()

    def fire_scatter(e, b):
        pltpu.async_copy(rows_v.at[b], acc.at[ed_v.at[e, 1]], ssem[b],
                         add=True)

    def drain_scatter(b):
        pltpu.make_async_copy(rows_v.at[b], acc.at[pl.ds(0, CHUNK)],
                              ssem[b]).wait()

    fire_ed(0, 0, 0)
    fire_ed(1, 1, 1)
    drain_ed(0)
    fire_gather(0, 0)

    def rnd(g, carry):
        for b in range(4):
            t = g * 4 + b
            rb = b % 2
            drain_gather(rb)

            def sgrp(q, cc):
                wv = w_v[b, pl.ds(q * 16, 16)]
                for l in range(16):
                    s = wv[l]
                    r = q * 16 + l
                    for k in range(D // 16):
                        rows_v[rb, r, pl.ds(k * 16, 16)] = (
                            rows_v[rb, r, pl.ds(k * 16, 16)] * s)
                return cc

            lax.fori_loop(0, CHUNK // 16, sgrp, 0)
            fire_scatter(b, rb)

            @pl.when(t < CPT - 1)
            def _():
                @pl.when(t >= 1)
                def _():
                    drain_scatter((b + 1) % 2)
                drain_ed((b + 1) % 2)
                fire_gather((b + 1) % 4, (b + 1) % 2)

            @pl.when(t < CPT - 2)
            def _():
                fire_ed(t + 2, (b + 2) % 4, b % 2)
        return carry

    lax.fori_loop(0, CPT // 4, rnd, 0)
    drain_scatter(0)
    drain_scatter(1)

    plsc.subcore_barrier()
    for t in range(STRIPE // CHUNK):
        pltpu.sync_copy(acc.at[pl.ds(r0 + t * CHUNK, CHUNK)],
                        out_hbm.at[cid, pl.ds(r0 + t * CHUNK, CHUNK)])


_R = 400
_GRID = (N_NODES // _R,)
_INV_SQRT2 = 0.7071067811865476


def _gelu(t):
    return 0.5 * t * (1.0 + lax.erf(t * _INV_SQRT2))


def _pre_body(x_ref, w_ref, d0_ref, d1_ref, y_ref, dis_ref):
    deg = d0_ref[...] + d1_ref[...] + 1.0
    dis = lax.rsqrt(deg)
    xw = jnp.dot(x_ref[...], w_ref[...], preferred_element_type=jnp.float32)
    y_ref[...] = xw * dis
    dis_ref[...] = dis


def _tc_pre(x, W1, d0, d1):
    return pl.pallas_call(
        _pre_body,
        grid=_GRID,
        in_specs=[
            pl.BlockSpec((_R, D), lambda i: (i, 0)),
            pl.BlockSpec((D, D), lambda i: (0, 0)),
            pl.BlockSpec((_R, 1), lambda i: (i, 0)),
            pl.BlockSpec((_R, 1), lambda i: (i, 0)),
        ],
        out_specs=[
            pl.BlockSpec((_R, D), lambda i: (i, 0)),
            pl.BlockSpec((_R, 1), lambda i: (i, 0)),
        ],
        out_shape=[
            jax.ShapeDtypeStruct((N_NODES, D), jnp.float32),
            jax.ShapeDtypeStruct((N_NODES, 1), jnp.float32),
        ],
    )(x, W1, d0, d1)


def _mid_body(p0_ref, p1_ref, y_ref, dis_ref, b_ref, w2_ref, y2_ref):
    dis = dis_ref[...]
    t = dis * (p0_ref[...] + p1_ref[...] + y_ref[...]) + b_ref[...]
    h = _gelu(t)
    y2_ref[...] = jnp.dot(h, w2_ref[...],
                          preferred_element_type=jnp.float32) * dis


def _tc_mid(p0, p1, y1, dis, b1, W2):
    return pl.pallas_call(
        _mid_body,
        grid=_GRID,
        in_specs=[
            pl.BlockSpec((_R, D), lambda i: (i, 0)),
            pl.BlockSpec((_R, D), lambda i: (i, 0)),
            pl.BlockSpec((_R, D), lambda i: (i, 0)),
            pl.BlockSpec((_R, 1), lambda i: (i, 0)),
            pl.BlockSpec((1, D), lambda i: (0, 0)),
            pl.BlockSpec((D, D), lambda i: (0, 0)),
        ],
        out_specs=pl.BlockSpec((_R, D), lambda i: (i, 0)),
        out_shape=jax.ShapeDtypeStruct((N_NODES, D), jnp.float32),
    )(p0, p1, y1, dis, b1, W2)


def _post_body(q0_ref, q1_ref, y2_ref, dis_ref, b_ref, out_ref):
    t = dis_ref[...] * (q0_ref[...] + q1_ref[...] + y2_ref[...]) + b_ref[...]
    out_ref[...] = _gelu(t)


def _tc_post(q0, q1, y2, dis, b2):
    return pl.pallas_call(
        _post_body,
        grid=_GRID,
        in_specs=[
            pl.BlockSpec((_R, D), lambda i: (i, 0)),
            pl.BlockSpec((_R, D), lambda i: (i, 0)),
            pl.BlockSpec((_R, D), lambda i: (i, 0)),
            pl.BlockSpec((_R, 1), lambda i: (i, 0)),
            pl.BlockSpec((1, D), lambda i: (0, 0)),
        ],
        out_specs=pl.BlockSpec((_R, D), lambda i: (i, 0)),
        out_shape=jax.ShapeDtypeStruct((N_NODES, D), jnp.float32),
    )(q0, q1, y2, dis, b2)


def kernel(x, edge_index, edge_weight, W1, b1, W2, b2):
    src = edge_index[0].astype(jnp.int32)
    dst = edge_index[1].astype(jnp.int32)
    w = edge_weight.astype(jnp.float32)
    pad = E_PAD - N_EDGES
    pad_idx = jnp.arange(pad, dtype=jnp.int32)
    src = jnp.concatenate([src, pad_idx % N_NODES])
    dst = jnp.concatenate([dst, pad_idx % N_PAD])
    w = jnp.concatenate([w, jnp.zeros((pad,), jnp.float32)])
    src = src.reshape(NW * CPT, CHUNK)
    dst = dst.reshape(NW * CPT, CHUNK)
    w = w.reshape(NW * CPT, CHUNK)
    ed = jnp.stack([src, dst], axis=1)

    degp = _sc_degree(dst, w)
    d0 = degp[0, :, None]
    d1 = degp[1, :, None]

    y1, dis = _tc_pre(x, W1, d0, d1)

    p = _sc_aggregate(y1, ed, w)
    y2 = _tc_mid(p[0], p[1], y1, dis,
                 b1.reshape(1, D), W2)

    q = _sc_aggregate(y2, ed, w)
    return _tc_post(q[0], q[1], y2, dis, b2.reshape(1, D))


# speedup vs baseline: 2.5285x; 1.0007x over previous
import functools

import jax
import jax.numpy as jnp
from jax import lax
from jax.experimental import pallas as pl
from jax.experimental.pallas import tpu as pltpu
from jax.experimental.pallas import tpu_sc as plsc

N_NODES = 10000
D = 128
N_EDGES = 320000

NC = 2
NS = 16
NW = NC * NS
CHUNK = 128
CPT = 80
NBUF = 4
E_PAD = NW * CPT * CHUNK
N_PAD = 10240
STRIPE = N_PAD // NS

_MESH = plsc.VectorSubcoreMesh(core_axis_name="c", subcore_axis_name="s")


def _zero_vmem_rows(ref, nrows):
    zrow = jnp.zeros((16,), jnp.float32)

    def zr(i, c):
        for k in range(D // 16):
            ref[i, pl.ds(k * 16, 16)] = zrow
        return c

    lax.fori_loop(0, nrows, zr, 0)


@functools.partial(
    pl.kernel,
    out_type=jax.ShapeDtypeStruct((NC, N_PAD), jnp.float32),
    mesh=_MESH,
    scratch_types=[
        pltpu.VMEM((CPT, CHUNK), jnp.int32),
        pltpu.VMEM((CPT, CHUNK), jnp.float32),
        pltpu.VMEM((STRIPE,), jnp.float32),
        pltpu.VMEM_SHARED((N_PAD,), jnp.float32),
        pltpu.SemaphoreType.DMA,
    ],
)
def _sc_degree(dst_hbm, w_hbm, out_hbm, dst_v, w_v, z_v, acc, sem):
    cid = lax.axis_index("c")
    sid = lax.axis_index("s")
    wid = cid * NS + sid

    zrow = jnp.zeros((16,), jnp.float32)
    for k in range(STRIPE // 16):
        z_v[pl.ds(k * 16, 16)] = zrow
    pltpu.sync_copy(z_v, acc.at[pl.ds(sid * STRIPE, STRIPE)])
    pltpu.sync_copy(dst_hbm.at[pl.ds(wid * CPT, CPT)], dst_v)
    pltpu.sync_copy(w_hbm.at[pl.ds(wid * CPT, CPT)], w_v)
    plsc.subcore_barrier()

    def fire(c, carry):
        pltpu.async_copy(w_v.at[c], acc.at[dst_v.at[c]], sem, add=True)
        return carry

    lax.fori_loop(0, CPT, fire, 0)

    def drain(c, carry):
        pltpu.make_async_copy(w_v.at[0], acc.at[pl.ds(0, CHUNK)], sem).wait()
        return carry

    lax.fori_loop(0, CPT, drain, 0)

    plsc.subcore_barrier()
    pltpu.sync_copy(acc.at[pl.ds(sid * STRIPE, STRIPE)],
                    out_hbm.at[cid, pl.ds(sid * STRIPE, STRIPE)])


@functools.partial(
    pl.kernel,
    out_type=jax.ShapeDtypeStruct((NC, N_PAD, D), jnp.float32),
    mesh=_MESH,
    scratch_types=[
        pltpu.VMEM((4, 2, CHUNK), jnp.int32),
        pltpu.VMEM((4, CHUNK), jnp.float32),
        pltpu.VMEM((2, CHUNK, D), jnp.float32),
        pltpu.VMEM_SHARED((N_PAD, D), jnp.float32),
        [pltpu.SemaphoreType.DMA] * 2,
        [pltpu.SemaphoreType.DMA] * 2,
        [pltpu.SemaphoreType.DMA] * 2,
    ],
)
def _sc_aggregate(y_hbm, ed_hbm, w_hbm, out_hbm, ed_v, w_v, rows_v, acc,
                  esem, gsem, ssem):
    cid = lax.axis_index("c")
    sid = lax.axis_index("s")
    wid = cid * NS + sid
    r0 = sid * STRIPE
    ebase = wid * CPT

    _zero_vmem_rows(rows_v.at[0], CHUNK)
    for t in range(STRIPE // CHUNK):
        pltpu.sync_copy(rows_v.at[0], acc.at[pl.ds(r0 + t * CHUNK, CHUNK)])
    plsc.subcore_barrier()

    def fire_ed(t, e, s):
        pltpu.async_copy(ed_hbm.at[ebase + t], ed_v.at[e], esem[s])
        pltpu.async_copy(w_hbm.at[ebase + t], w_v.at[e], esem[s])

    def drain_ed(s):
        pltpu.make_async_copy(ed_hbm.at[0], ed_v.at[0], esem[s]).wait()
        pltpu.make_async_copy(w_hbm.at[0], w_v.at[0], esem[s]).wait()

    def fire_gather(e, b):
        pltpu.async_copy(y_hbm.at[ed_v.at[e, 0]], rows_v.at[b], gsem[b])

    def drain_gather(b):
        pltpu.make_async_copy(y_hbm.at[pl.ds(0, CHUNK)], rows_v.at[b],
                              gsem[b]).wait()

    def fire_scatter(e, b):
        pltpu.async_copy(rows_v.at[b], acc.at[ed_v.at[e, 1]], ssem[b],
                         add=True)

    def drain_scatter(b):
        pltpu.make_async_copy(rows_v.at[b], acc.at[pl.ds(0, CHUNK)],
                              ssem[b]).wait()

    fire_ed(0, 0, 0)
    fire_ed(1, 1, 1)
    drain_ed(0)
    fire_gather(0, 0)

    def rnd(g, carry):
        for b in range(4):
            t = g * 4 + b
            rb = b % 2
            drain_gather(rb)

            def sgrp(q, cc):
                wv = w_v[b, pl.ds(q * 16, 16)]
                for l in range(16):
                    s = wv[l]
                    r = q * 16 + l
                    for k in range(D // 16):
                        rows_v[rb, r, pl.ds(k * 16, 16)] = (
                            rows_v[rb, r, pl.ds(k * 16, 16)] * s)
                return cc

            lax.fori_loop(0, CHUNK // 16, sgrp, 0)

            @pl.when(t < CPT - 1)
            def _():
                @pl.when(t >= 1)
                def _():
                    drain_scatter((b + 1) % 2)
                drain_ed((b + 1) % 2)
                fire_gather((b + 1) % 4, (b + 1) % 2)

            fire_scatter(b, rb)

            @pl.when(t < CPT - 2)
            def _():
                fire_ed(t + 2, (b + 2) % 4, b % 2)
        return carry

    lax.fori_loop(0, CPT // 4, rnd, 0)
    drain_scatter(0)
    drain_scatter(1)

    plsc.subcore_barrier()
    for t in range(STRIPE // CHUNK):
        pltpu.sync_copy(acc.at[pl.ds(r0 + t * CHUNK, CHUNK)],
                        out_hbm.at[cid, pl.ds(r0 + t * CHUNK, CHUNK)])


_R = 400
_GRID = (N_NODES // _R,)
_INV_SQRT2 = 0.7071067811865476


def _gelu(t):
    return 0.5 * t * (1.0 + lax.erf(t * _INV_SQRT2))


def _pre_body(x_ref, w_ref, d0_ref, d1_ref, y_ref, dis_ref):
    deg = d0_ref[...] + d1_ref[...] + 1.0
    dis = lax.rsqrt(deg)
    xw = jnp.dot(x_ref[...], w_ref[...], preferred_element_type=jnp.float32)
    y_ref[...] = xw * dis
    dis_ref[...] = dis


def _tc_pre(x, W1, d0, d1):
    return pl.pallas_call(
        _pre_body,
        grid=_GRID,
        in_specs=[
            pl.BlockSpec((_R, D), lambda i: (i, 0)),
            pl.BlockSpec((D, D), lambda i: (0, 0)),
            pl.BlockSpec((_R, 1), lambda i: (i, 0)),
            pl.BlockSpec((_R, 1), lambda i: (i, 0)),
        ],
        out_specs=[
            pl.BlockSpec((_R, D), lambda i: (i, 0)),
            pl.BlockSpec((_R, 1), lambda i: (i, 0)),
        ],
        out_shape=[
            jax.ShapeDtypeStruct((N_NODES, D), jnp.float32),
            jax.ShapeDtypeStruct((N_NODES, 1), jnp.float32),
        ],
    )(x, W1, d0, d1)


def _mid_body(p0_ref, p1_ref, y_ref, dis_ref, b_ref, w2_ref, y2_ref):
    dis = dis_ref[...]
    t = dis * (p0_ref[...] + p1_ref[...] + y_ref[...]) + b_ref[...]
    h = _gelu(t)
    y2_ref[...] = jnp.dot(h, w2_ref[...],
                          preferred_element_type=jnp.float32) * dis


def _tc_mid(p0, p1, y1, dis, b1, W2):
    return pl.pallas_call(
        _mid_body,
        grid=_GRID,
        in_specs=[
            pl.BlockSpec((_R, D), lambda i: (i, 0)),
            pl.BlockSpec((_R, D), lambda i: (i, 0)),
            pl.BlockSpec((_R, D), lambda i: (i, 0)),
            pl.BlockSpec((_R, 1), lambda i: (i, 0)),
            pl.BlockSpec((1, D), lambda i: (0, 0)),
            pl.BlockSpec((D, D), lambda i: (0, 0)),
        ],
        out_specs=pl.BlockSpec((_R, D), lambda i: (i, 0)),
        out_shape=jax.ShapeDtypeStruct((N_NODES, D), jnp.float32),
    )(p0, p1, y1, dis, b1, W2)


def _post_body(q0_ref, q1_ref, y2_ref, dis_ref, b_ref, out_ref):
    t = dis_ref[...] * (q0_ref[...] + q1_ref[...] + y2_ref[...]) + b_ref[...]
    out_ref[...] = _gelu(t)


def _tc_post(q0, q1, y2, dis, b2):
    return pl.pallas_call(
        _post_body,
        grid=_GRID,
        in_specs=[
            pl.BlockSpec((_R, D), lambda i: (i, 0)),
            pl.BlockSpec((_R, D), lambda i: (i, 0)),
            pl.BlockSpec((_R, D), lambda i: (i, 0)),
            pl.BlockSpec((_R, 1), lambda i: (i, 0)),
            pl.BlockSpec((1, D), lambda i: (0, 0)),
        ],
        out_specs=pl.BlockSpec((_R, D), lambda i: (i, 0)),
        out_shape=jax.ShapeDtypeStruct((N_NODES, D), jnp.float32),
    )(q0, q1, y2, dis, b2)


def kernel(x, edge_index, edge_weight, W1, b1, W2, b2):
    src = edge_index[0].astype(jnp.int32)
    dst = edge_index[1].astype(jnp.int32)
    w = edge_weight.astype(jnp.float32)
    pad = E_PAD - N_EDGES
    pad_idx = jnp.arange(pad, dtype=jnp.int32)
    src = jnp.concatenate([src, pad_idx % N_NODES])
    dst = jnp.concatenate([dst, pad_idx % N_PAD])
    w = jnp.concatenate([w, jnp.zeros((pad,), jnp.float32)])
    src = src.reshape(NW * CPT, CHUNK)
    dst = dst.reshape(NW * CPT, CHUNK)
    w = w.reshape(NW * CPT, CHUNK)
    ed = jnp.stack([src, dst], axis=1)

    degp = _sc_degree(dst, w)
    d0 = degp[0, :, None]
    d1 = degp[1, :, None]

    y1, dis = _tc_pre(x, W1, d0, d1)

    p = _sc_aggregate(y1, ed, w)
    y2 = _tc_mid(p[0], p[1], y1, dis,
                 b1.reshape(1, D), W2)

    q = _sc_aggregate(y2, ed, w)
    return _tc_post(q[0], q[1], y2, dis, b2.reshape(1, D))


# speedup vs baseline: 3.0743x; 1.2158x over previous
import functools

import jax
import jax.numpy as jnp
from jax import lax
from jax.experimental import pallas as pl
from jax.experimental.pallas import tpu as pltpu
from jax.experimental.pallas import tpu_sc as plsc

N_NODES = 10000
D = 128
N_EDGES = 320000

NC = 2
NS = 16
NW = NC * NS
CHUNK = 128
CPT = 80
NBUF = 4
E_PAD = NW * CPT * CHUNK
N_PAD = 10240
STRIPE = N_PAD // NS

_MESH = plsc.VectorSubcoreMesh(core_axis_name="c", subcore_axis_name="s")


def _zero_vmem_rows(ref, nrows):
    zrow = jnp.zeros((16,), jnp.float32)

    def zr(i, c):
        for k in range(D // 16):
            ref[i, pl.ds(k * 16, 16)] = zrow
        return c

    lax.fori_loop(0, nrows, zr, 0)


@functools.partial(
    pl.kernel,
    out_type=jax.ShapeDtypeStruct((NC, N_PAD), jnp.float32),
    mesh=_MESH,
    scratch_types=[
        pltpu.VMEM((CPT, CHUNK), jnp.int32),
        pltpu.VMEM((CPT, CHUNK), jnp.float32),
        pltpu.VMEM((STRIPE,), jnp.float32),
        pltpu.VMEM_SHARED((N_PAD,), jnp.float32),
        pltpu.SemaphoreType.DMA,
    ],
)
def _sc_degree(dst_hbm, w_hbm, out_hbm, dst_v, w_v, z_v, acc, sem):
    cid = lax.axis_index("c")
    sid = lax.axis_index("s")
    wid = cid * NS + sid

    zrow = jnp.zeros((16,), jnp.float32)
    for k in range(STRIPE // 16):
        z_v[pl.ds(k * 16, 16)] = zrow
    pltpu.sync_copy(z_v, acc.at[pl.ds(sid * STRIPE, STRIPE)])
    pltpu.sync_copy(dst_hbm.at[pl.ds(wid * CPT, CPT)], dst_v)
    pltpu.sync_copy(w_hbm.at[pl.ds(wid * CPT, CPT)], w_v)
    plsc.subcore_barrier()

    def fire(c, carry):
        pltpu.async_copy(w_v.at[c], acc.at[dst_v.at[c]], sem, add=True)
        return carry

    lax.fori_loop(0, CPT, fire, 0)

    def drain(c, carry):
        pltpu.make_async_copy(w_v.at[0], acc.at[pl.ds(0, CHUNK)], sem).wait()
        return carry

    lax.fori_loop(0, CPT, drain, 0)

    plsc.subcore_barrier()
    pltpu.sync_copy(acc.at[pl.ds(sid * STRIPE, STRIPE)],
                    out_hbm.at[cid, pl.ds(sid * STRIPE, STRIPE)])


@functools.partial(
    pl.kernel,
    out_type=jax.ShapeDtypeStruct((NC, N_PAD, D), jnp.float32),
    mesh=_MESH,
    scratch_types=[
        pltpu.VMEM((4, 2, CHUNK), jnp.int32),
        pltpu.VMEM((4, CHUNK), jnp.float32),
        pltpu.VMEM((2, CHUNK, D), jnp.float32),
        pltpu.VMEM_SHARED((N_PAD, D), jnp.float32),
        [pltpu.SemaphoreType.DMA] * 2,
        [pltpu.SemaphoreType.DMA] * 2,
        [pltpu.SemaphoreType.DMA] * 2,
    ],
)
def _sc_aggregate(y_hbm, ed_hbm, w_hbm, out_hbm, ed_v, w_v, rows_v, acc,
                  esem, gsem, ssem):
    cid = lax.axis_index("c")
    sid = lax.axis_index("s")
    wid = cid * NS + sid
    r0 = sid * STRIPE
    ebase = wid * CPT

    _zero_vmem_rows(rows_v.at[0], CHUNK)
    for t in range(STRIPE // CHUNK):
        pltpu.sync_copy(rows_v.at[0], acc.at[pl.ds(r0 + t * CHUNK, CHUNK)])
    plsc.subcore_barrier()

    def fire_ed(t, e, s):
        pltpu.async_copy(ed_hbm.at[ebase + t], ed_v.at[e], esem[s])
        pltpu.async_copy(w_hbm.at[ebase + t], w_v.at[e], esem[s])

    def drain_ed(s):
        pltpu.make_async_copy(ed_hbm.at[0], ed_v.at[0], esem[s]).wait()
        pltpu.make_async_copy(w_hbm.at[0], w_v.at[0], esem[s]).wait()

    def fire_gather(e, b):
        pltpu.async_copy(y_hbm.at[ed_v.at[e, 0]], rows_v.at[b], gsem[b])

    def drain_gather(b):
        pltpu.make_async_copy(y_hbm.at[pl.ds(0, CHUNK)], rows_v.at[b],
                              gsem[b]).wait()

    def fire_scatter(e, b):
        pltpu.async_copy(rows_v.at[b], acc.at[ed_v.at[e, 1]], ssem[b],
                         add=True)

    def drain_scatter(b):
        pltpu.make_async_copy(rows_v.at[b], acc.at[pl.ds(0, CHUNK)],
                              ssem[b]).wait()

    fire_ed(0, 0, 0)
    fire_ed(1, 1, 1)
    drain_ed(0)
    fire_gather(0, 0)

    def rnd(g, carry):
        for b in range(4):
            t = g * 4 + b
            rb = b % 2
            drain_gather(rb)

            @pl.when(t < CPT - 1)
            def _():
                @pl.when(t >= 1)
                def _():
                    drain_scatter((b + 1) % 2)
                drain_ed((b + 1) % 2)
                fire_gather((b + 1) % 4, (b + 1) % 2)

            def sgrp(q, cc):
                wv = w_v[b, pl.ds(q * 16, 16)]
                for l in range(16):
                    s = wv[l]
                    r = q * 16 + l
                    for k in range(D // 16):
                        rows_v[rb, r, pl.ds(k * 16, 16)] = (
                            rows_v[rb, r, pl.ds(k * 16, 16)] * s)
                return cc

            lax.fori_loop(0, CHUNK // 16, sgrp, 0)
            fire_scatter(b, rb)

            @pl.when(t < CPT - 2)
            def _():
                fire_ed(t + 2, (b + 2) % 4, b % 2)
        return carry

    lax.fori_loop(0, CPT // 4, rnd, 0)
    drain_scatter(0)
    drain_scatter(1)

    plsc.subcore_barrier()
    for t in range(STRIPE // CHUNK):
        pltpu.sync_copy(acc.at[pl.ds(r0 + t * CHUNK, CHUNK)],
                        out_hbm.at[cid, pl.ds(r0 + t * CHUNK, CHUNK)])


_R = 400
_GRID = (N_NODES // _R,)
_INV_SQRT2 = 0.7071067811865476


def _gelu(t):
    return 0.5 * t * (1.0 + lax.erf(t * _INV_SQRT2))


def _pre_body(x_ref, w_ref, d0_ref, d1_ref, y_ref, dis_ref):
    deg = d0_ref[...] + d1_ref[...] + 1.0
    dis = lax.rsqrt(deg)
    xw = jnp.dot(x_ref[...], w_ref[...], preferred_element_type=jnp.float32)
    y_ref[...] = xw * dis
    dis_ref[...] = dis


def _tc_pre(x, W1, d0, d1):
    return pl.pallas_call(
        _pre_body,
        grid=_GRID,
        in_specs=[
            pl.BlockSpec((_R, D), lambda i: (i, 0)),
            pl.BlockSpec((D, D), lambda i: (0, 0)),
            pl.BlockSpec((_R, 1), lambda i: (i, 0)),
            pl.BlockSpec((_R, 1), lambda i: (i, 0)),
        ],
        out_specs=[
            pl.BlockSpec((_R, D), lambda i: (i, 0)),
            pl.BlockSpec((_R, 1), lambda i: (i, 0)),
        ],
        out_shape=[
            jax.ShapeDtypeStruct((N_NODES, D), jnp.float32),
            jax.ShapeDtypeStruct((N_NODES, 1), jnp.float32),
        ],
    )(x, W1, d0, d1)


def _mid_body(p0_ref, p1_ref, y_ref, dis_ref, b_ref, w2_ref, y2_ref):
    dis = dis_ref[...]
    t = dis * (p0_ref[...] + p1_ref[...] + y_ref[...]) + b_ref[...]
    h = _gelu(t)
    y2_ref[...] = jnp.dot(h, w2_ref[...],
                          preferred_element_type=jnp.float32) * dis


def _tc_mid(p0, p1, y1, dis, b1, W2):
    return pl.pallas_call(
        _mid_body,
        grid=_GRID,
        in_specs=[
            pl.BlockSpec((_R, D), lambda i: (i, 0)),
            pl.BlockSpec((_R, D), lambda i: (i, 0)),
            pl.BlockSpec((_R, D), lambda i: (i, 0)),
            pl.BlockSpec((_R, 1), lambda i: (i, 0)),
            pl.BlockSpec((1, D), lambda i: (0, 0)),
            pl.BlockSpec((D, D), lambda i: (0, 0)),
        ],
        out_specs=pl.BlockSpec((_R, D), lambda i: (i, 0)),
        out_shape=jax.ShapeDtypeStruct((N_NODES, D), jnp.float32),
    )(p0, p1, y1, dis, b1, W2)


def _post_body(q0_ref, q1_ref, y2_ref, dis_ref, b_ref, out_ref):
    t = dis_ref[...] * (q0_ref[...] + q1_ref[...] + y2_ref[...]) + b_ref[...]
    out_ref[...] = _gelu(t)


def _tc_post(q0, q1, y2, dis, b2):
    return pl.pallas_call(
        _post_body,
        grid=_GRID,
        in_specs=[
            pl.BlockSpec((_R, D), lambda i: (i, 0)),
            pl.BlockSpec((_R, D), lambda i: (i, 0)),
            pl.BlockSpec((_R, D), lambda i: (i, 0)),
            pl.BlockSpec((_R, 1), lambda i: (i, 0)),
            pl.BlockSpec((1, D), lambda i: (0, 0)),
        ],
        out_specs=pl.BlockSpec((_R, D), lambda i: (i, 0)),
        out_shape=jax.ShapeDtypeStruct((N_NODES, D), jnp.float32),
    )(q0, q1, y2, dis, b2)


def kernel(x, edge_index, edge_weight, W1, b1, W2, b2):
    src = edge_index[0].astype(jnp.int32)
    dst = edge_index[1].astype(jnp.int32)
    w = edge_weight.astype(jnp.float32)
    pad = E_PAD - N_EDGES
    pad_idx = jnp.arange(pad, dtype=jnp.int32)
    src = jnp.concatenate([src, pad_idx % N_NODES])
    dst = jnp.concatenate([dst, pad_idx % N_PAD])
    w = jnp.concatenate([w, jnp.zeros((pad,), jnp.float32)])
    src = src.reshape(NW * CPT, CHUNK)
    dst = dst.reshape(NW * CPT, CHUNK)
    w = w.reshape(NW * CPT, CHUNK)
    ed = jnp.stack([src, dst], axis=1)

    degp = _sc_degree(dst, w)
    d0 = degp[0, :, None]
    d1 = degp[1, :, None]

    y1, dis = _tc_pre(x, W1, d0, d1)

    p = _sc_aggregate(y1, ed, w)
    y2 = _tc_mid(p[0], p[1], y1, dis,
                 b1.reshape(1, D), W2)

    q = _sc_aggregate(y2, ed, w)
    return _tc_post(q[0], q[1], y2, dis, b2.reshape(1, D))


# speedup vs baseline: 3.2030x; 1.0419x over previous
import functools

import jax
import jax.numpy as jnp
from jax import lax
from jax.experimental import pallas as pl
from jax.experimental.pallas import tpu as pltpu
from jax.experimental.pallas import tpu_sc as plsc

N_NODES = 10000
D = 128
N_EDGES = 320000

NC = 2
NS = 16
NW = NC * NS
CHUNK = 128
CPT = 80
NBUF = 4
E_PAD = NW * CPT * CHUNK
N_PAD = 10240
STRIPE = N_PAD // NS

_MESH = plsc.VectorSubcoreMesh(core_axis_name="c", subcore_axis_name="s")


def _zero_vmem_rows(ref, nrows):
    zrow = jnp.zeros((16,), jnp.float32)

    def zr(i, c):
        for k in range(D // 16):
            ref[i, pl.ds(k * 16, 16)] = zrow
        return c

    lax.fori_loop(0, nrows, zr, 0)


@functools.partial(
    pl.kernel,
    out_type=jax.ShapeDtypeStruct((NC, N_PAD), jnp.float32),
    mesh=_MESH,
    scratch_types=[
        pltpu.VMEM((CPT, CHUNK), jnp.int32),
        pltpu.VMEM((CPT, CHUNK), jnp.float32),
        pltpu.VMEM((STRIPE,), jnp.float32),
        pltpu.VMEM_SHARED((N_PAD,), jnp.float32),
        pltpu.SemaphoreType.DMA,
    ],
)
def _sc_degree(dst_hbm, w_hbm, out_hbm, dst_v, w_v, z_v, acc, sem):
    cid = lax.axis_index("c")
    sid = lax.axis_index("s")
    wid = cid * NS + sid

    zrow = jnp.zeros((16,), jnp.float32)
    for k in range(STRIPE // 16):
        z_v[pl.ds(k * 16, 16)] = zrow
    pltpu.sync_copy(z_v, acc.at[pl.ds(sid * STRIPE, STRIPE)])
    pltpu.sync_copy(dst_hbm.at[pl.ds(wid * CPT, CPT)], dst_v)
    pltpu.sync_copy(w_hbm.at[pl.ds(wid * CPT, CPT)], w_v)
    plsc.subcore_barrier()

    def fire(c, carry):
        pltpu.async_copy(w_v.at[c], acc.at[dst_v.at[c]], sem, add=True)
        return carry

    lax.fori_loop(0, CPT, fire, 0)

    def drain(c, carry):
        pltpu.make_async_copy(w_v.at[0], acc.at[pl.ds(0, CHUNK)], sem).wait()
        return carry

    lax.fori_loop(0, CPT, drain, 0)

    plsc.subcore_barrier()
    pltpu.sync_copy(acc.at[pl.ds(sid * STRIPE, STRIPE)],
                    out_hbm.at[cid, pl.ds(sid * STRIPE, STRIPE)])


@functools.partial(
    pl.kernel,
    out_type=jax.ShapeDtypeStruct((NC, N_PAD, D), jnp.float32),
    mesh=_MESH,
    scratch_types=[
        pltpu.VMEM((4, 2, CHUNK), jnp.int32),
        pltpu.VMEM((4, CHUNK), jnp.float32),
        pltpu.VMEM((2, CHUNK, D), jnp.float32),
        pltpu.VMEM_SHARED((N_PAD, D), jnp.float32),
        [pltpu.SemaphoreType.DMA] * 2,
        [pltpu.SemaphoreType.DMA] * 2,
        [pltpu.SemaphoreType.DMA] * 2,
    ],
)
def _sc_aggregate(y_hbm, ed_hbm, w_hbm, out_hbm, ed_v, w_v, rows_v, acc,
                  esem, gsem, ssem):
    cid = lax.axis_index("c")
    sid = lax.axis_index("s")
    wid = cid * NS + sid
    r0 = sid * STRIPE
    ebase = wid * CPT

    _zero_vmem_rows(rows_v.at[0], CHUNK)
    for t in range(STRIPE // CHUNK):
        pltpu.sync_copy(rows_v.at[0], acc.at[pl.ds(r0 + t * CHUNK, CHUNK)])
    plsc.subcore_barrier()

    def fire_ed(t, e, s):
        pltpu.async_copy(ed_hbm.at[ebase + t], ed_v.at[e], esem[s])
        pltpu.async_copy(w_hbm.at[ebase + t], w_v.at[e], esem[s])

    def drain_ed(s):
        pltpu.make_async_copy(ed_hbm.at[0], ed_v.at[0], esem[s]).wait()
        pltpu.make_async_copy(w_hbm.at[0], w_v.at[0], esem[s]).wait()

    def fire_gather(e, b):
        pltpu.async_copy(y_hbm.at[ed_v.at[e, 0]], rows_v.at[b], gsem[b])

    def drain_gather(b):
        pltpu.make_async_copy(y_hbm.at[pl.ds(0, CHUNK)], rows_v.at[b],
                              gsem[b]).wait()

    def fire_scatter(e, b):
        pltpu.async_copy(rows_v.at[b], acc.at[ed_v.at[e, 1]], ssem[b],
                         add=True)

    def drain_scatter(b):
        pltpu.make_async_copy(rows_v.at[b], acc.at[pl.ds(0, CHUNK)],
                              ssem[b]).wait()

    fire_ed(0, 0, 0)
    fire_ed(1, 1, 1)
    drain_ed(0)
    fire_gather(0, 0)

    def rnd(g, carry):
        for b in range(4):
            t = g * 4 + b
            rb = b % 2
            drain_gather(rb)

            @pl.when(t < CPT - 1)
            def _():
                @pl.when(t >= 1)
                def _():
                    drain_scatter((b + 1) % 2)
                drain_ed((b + 1) % 2)
                fire_gather((b + 1) % 4, (b + 1) % 2)

            def sgrp(q, cc):
                wv = w_v[b, pl.ds(q * 16, 16)]
                for l in range(16):
                    s = wv[l]
                    r = q * 16 + l
                    for k in range(D // 16):
                        rows_v[rb, r, pl.ds(k * 16, 16)] = (
                            rows_v[rb, r, pl.ds(k * 16, 16)] * s)
                return cc

            lax.fori_loop(0, CHUNK // 16, sgrp, 0)
            fire_scatter(b, rb)

            @pl.when(t < CPT - 2)
            def _():
                fire_ed(t + 2, (b + 2) % 4, b % 2)
        return carry

    lax.fori_loop(0, CPT // 4, rnd, 0)
    drain_scatter(0)
    drain_scatter(1)

    plsc.subcore_barrier()
    for t in range(STRIPE // CHUNK):
        pltpu.sync_copy(acc.at[pl.ds(r0 + t * CHUNK, CHUNK)],
                        out_hbm.at[cid, pl.ds(r0 + t * CHUNK, CHUNK)])


_R = 400
_GRID = (N_NODES // _R,)
_INV_SQRT2 = 0.7071067811865476


def _gelu(t):
    return 0.5 * t * (1.0 + lax.erf(t * _INV_SQRT2))


def _pre_body(x_ref, w_ref, d0_ref, d1_ref, y_ref, dis_ref):
    deg = d0_ref[0] + d1_ref[0] + 1.0
    dis = lax.rsqrt(deg)
    xw = jnp.dot(x_ref[...], w_ref[...], preferred_element_type=jnp.float32)
    y_ref[...] = xw * dis
    dis_ref[...] = dis


def _tc_pre(x, W1, d0, d1):
    return pl.pallas_call(
        _pre_body,
        grid=_GRID,
        in_specs=[
            pl.BlockSpec((_R, D), lambda i: (i, 0)),
            pl.BlockSpec((D, D), lambda i: (0, 0)),
            pl.BlockSpec((1, _R, 1), lambda i: (0, i, 0)),
            pl.BlockSpec((1, _R, 1), lambda i: (1, i, 0)),
        ],
        out_specs=[
            pl.BlockSpec((_R, D), lambda i: (i, 0)),
            pl.BlockSpec((_R, 1), lambda i: (i, 0)),
        ],
        out_shape=[
            jax.ShapeDtypeStruct((N_NODES, D), jnp.float32),
            jax.ShapeDtypeStruct((N_NODES, 1), jnp.float32),
        ],
    )(x, W1, d0, d1)


def _mid_body(p0_ref, p1_ref, y_ref, dis_ref, b_ref, w2_ref, y2_ref):
    dis = dis_ref[...]
    t = dis * (p0_ref[0] + p1_ref[0] + y_ref[...]) + b_ref[...]
    h = _gelu(t)
    y2_ref[...] = jnp.dot(h, w2_ref[...],
                          preferred_element_type=jnp.float32) * dis


def _tc_mid(p0, p1, y1, dis, b1, W2):
    return pl.pallas_call(
        _mid_body,
        grid=_GRID,
        in_specs=[
            pl.BlockSpec((1, _R, D), lambda i: (0, i, 0)),
            pl.BlockSpec((1, _R, D), lambda i: (1, i, 0)),
            pl.BlockSpec((_R, D), lambda i: (i, 0)),
            pl.BlockSpec((_R, 1), lambda i: (i, 0)),
            pl.BlockSpec((1, D), lambda i: (0, 0)),
            pl.BlockSpec((D, D), lambda i: (0, 0)),
        ],
        out_specs=pl.BlockSpec((_R, D), lambda i: (i, 0)),
        out_shape=jax.ShapeDtypeStruct((N_NODES, D), jnp.float32),
    )(p0, p1, y1, dis, b1, W2)


def _post_body(q0_ref, q1_ref, y2_ref, dis_ref, b_ref, out_ref):
    t = dis_ref[...] * (q0_ref[0] + q1_ref[0] + y2_ref[...]) + b_ref[...]
    out_ref[...] = _gelu(t)


def _tc_post(q0, q1, y2, dis, b2):
    return pl.pallas_call(
        _post_body,
        grid=_GRID,
        in_specs=[
            pl.BlockSpec((1, _R, D), lambda i: (0, i, 0)),
            pl.BlockSpec((1, _R, D), lambda i: (1, i, 0)),
            pl.BlockSpec((_R, D), lambda i: (i, 0)),
            pl.BlockSpec((_R, 1), lambda i: (i, 0)),
            pl.BlockSpec((1, D), lambda i: (0, 0)),
        ],
        out_specs=pl.BlockSpec((_R, D), lambda i: (i, 0)),
        out_shape=jax.ShapeDtypeStruct((N_NODES, D), jnp.float32),
    )(q0, q1, y2, dis, b2)


def kernel(x, edge_index, edge_weight, W1, b1, W2, b2):
    src = edge_index[0].astype(jnp.int32)
    dst = edge_index[1].astype(jnp.int32)
    w = edge_weight.astype(jnp.float32)
    pad = E_PAD - N_EDGES
    pad_idx = jnp.arange(pad, dtype=jnp.int32)
    src = jnp.concatenate([src, pad_idx % N_NODES])
    dst = jnp.concatenate([dst, pad_idx % N_PAD])
    w = jnp.concatenate([w, jnp.zeros((pad,), jnp.float32)])
    src = src.reshape(NW * CPT, CHUNK)
    dst = dst.reshape(NW * CPT, CHUNK)
    w = w.reshape(NW * CPT, CHUNK)
    ed = jnp.stack([src, dst], axis=1)

    degp = _sc_degree(dst, w)[:, :, None]

    y1, dis = _tc_pre(x, W1, degp, degp)

    p = _sc_aggregate(y1, ed, w)
    y2 = _tc_mid(p, p, y1, dis,
                 b1.reshape(1, D), W2)

    q = _sc_aggregate(y2, ed, w)
    return _tc_post(q, q, y2, dis, b2.reshape(1, D))
